# baseline (speedup 1.0000x reference)
import functools

import jax
import jax.numpy as jnp
from jax import lax
from jax.experimental import pallas as pl
from jax.experimental.pallas import tpu as pltpu
from jax.experimental.pallas import tpu_sc as plsc

E = 64
BK = 1024
B = 4096
NTOT = 6 * B
K_TIME = 168
K_CAT = 400
CHUNK = 128
NBUF = 4
BP = 2000
NCH = 6
CB = NTOT // NCH
CBLK = CB // BK


def _pack_tables(user_time_W, user_cat_W, POI_time_W, POI_cat_W):
    a = jnp.concatenate([user_time_W, user_cat_W], axis=1)
    b = jnp.concatenate([POI_time_W, POI_cat_W], axis=1)
    abits = lax.bitcast_convert_type(a, jnp.int32) + 0x8000
    bbits = lax.bitcast_convert_type(b, jnp.int32) + 0x8000
    hi = jnp.bitwise_and(abits, -65536)
    lo = lax.shift_right_logical(bbits, 16)
    return jnp.bitwise_or(hi, lo)


def _sc_gather(all4, idx_all, base_row, n_rows):
    info = plsc.get_sparse_core_info()
    nw = info.num_cores * info.num_subcores
    bpw = n_rows // nw
    nchunk = bpw // CHUNK
    mesh = plsc.VectorSubcoreMesh(core_axis_name="c", subcore_axis_name="s")
    scratch = [pltpu.VMEM((CHUNK,), jnp.int32) for _ in range(NBUF)]
    scratch += [pltpu.VMEM((NBUF, CHUNK, 2 * E), jnp.int32),
                pltpu.SemaphoreType.DMA]

    @functools.partial(
        pl.kernel, mesh=mesh,
        out_type=jax.ShapeDtypeStruct((n_rows, 2 * E), jnp.int32),
        scratch_types=scratch)
    def gk(tab, idx, out, *sc):
        idxv = sc[:NBUF]
        rows = sc[NBUF]
        sem = sc[NBUF + 1]
        wid = lax.axis_index("s") * info.num_cores + lax.axis_index("c")
        base = wid * bpw
        descs = [None] * nchunk

        def writeout(k):
            pltpu.sync_copy(rows.at[k % NBUF],
                            out.at[pl.ds(base + k * CHUNK, CHUNK)])

        for k in range(nchunk):
            if k >= NBUF:
                descs[k - NBUF].wait()
                writeout(k - NBUF)
            pltpu.sync_copy(
                idx.at[pl.ds(base_row + base + k * CHUNK, CHUNK)],
                idxv[k % NBUF])
            descs[k] = pltpu.async_copy(tab.at[idxv[k % NBUF]],
                                        rows.at[k % NBUF], sem)
        for k in range(nchunk - NBUF, nchunk):
            descs[k].wait()
            writeout(k)

    return gk(all4, idx_all)


def _log_sigmoid(x):
    return -(jnp.maximum(-x, 0.0) + jnp.log(1.0 + jnp.exp(-jnp.abs(x))))


def _tc_score_chunk(c, tcol, ccol, g, WtT, WcT, trtT, trcT, rep, tile):
    bps = B // BK
    has_pos = c == 0

    def body(t_ref, c_ref, gua_ref, gub_ref, gpa_ref, gpb_ref,
             wtT_ref, wcT_ref, trtT_ref, trcT_ref, rep_ref, tile_ref,
             *out_refs):
        jl = pl.program_id(0)
        jg = c * CBLK + jl
        if has_pos:
            pos_ref, neg_ref = out_refs
        else:
            (neg_ref,) = out_refs

        def score(u, p, wT, trT, idx, K):
            prep = jnp.dot(p, rep_ref[...], preferred_element_type=jnp.float32)
            u2 = jnp.concatenate([u, u], axis=1)
            v = jnp.concatenate(
                [(prep[:, k * 128:(k + 1) * 128] * u2).astype(jnp.bfloat16)
                 for k in range(E * E // 128)], axis=1)
            cdims = (((1,), (1,)), ((), ()))
            y = lax.dot_general(v, wT, cdims,
                                preferred_element_type=jnp.float32)
            y = y + jnp.dot(p, trT, preferred_element_type=jnp.float32)
            oh = (lax.broadcasted_iota(jnp.int32, (BK, K), 1) == idx)
            return jnp.sum(jnp.where(oh, y, 0.0), axis=1, keepdims=True)

        def unpack_user(x):
            return lax.bitcast_convert_type(
                jnp.bitwise_and(x, -65536), jnp.float32).astype(jnp.bfloat16)

        def unpack_poi(x):
            return lax.bitcast_convert_type(
                lax.shift_left(x, 16), jnp.float32).astype(jnp.bfloat16)

        ut = unpack_user(gua_ref[:, :E])
        uc = unpack_user(gub_ref[:, E:])
        pt = unpack_poi(gpa_ref[:, :E])
        pc = unpack_poi(gpb_ref[:, E:])
        s_t = score(ut, pt, wtT_ref[...], trtT_ref[...], t_ref[...], K_TIME)
        s_c = score(uc, pc, wcT_ref[...], trcT_ref[...], c_ref[...], K_CAT)
        set_id = jg // bps

        @pl.when(jl == 0)
        def _():
            neg_ref[...] = jnp.zeros_like(neg_ref)

        if has_pos:
            @pl.when(set_id == 0)
            def _():
                pos_ref[...] = -(_log_sigmoid(s_t) + _log_sigmoid(s_c))

        @pl.when(set_id > 0)
        def _():
            part = jnp.sum(_log_sigmoid(-s_t) + _log_sigmoid(-s_c))
            rr = lax.broadcasted_iota(jnp.int32, (8, 128), 0)
            cc = lax.broadcasted_iota(jnp.int32, (8, 128), 1)
            m = (rr == (set_id - 1)) & (cc == 0)
            neg_ref[...] = neg_ref[...] + jnp.where(m, -part, 0.0)

    const = lambda j: (0, 0)
    srow = lambda j: (c * CBLK + j, 0)
    out_specs = [pl.BlockSpec((8, 128), const)]
    out_shape = [jax.ShapeDtypeStruct((8, 128), jnp.float32)]
    if has_pos:
        out_specs = [pl.BlockSpec((BK, 1), lambda j: (jnp.minimum(j, bps - 1), 0))] + out_specs
        out_shape = [jax.ShapeDtypeStruct((B, 1), jnp.float32)] + out_shape
    return pl.pallas_call(
        body,
        grid=(CBLK,),
        in_specs=[
            pl.BlockSpec((BK, 1), srow),
            pl.BlockSpec((BK, 1), srow),
            pl.BlockSpec((BK, 2 * E), lambda j: (j, 0)),
            pl.BlockSpec((BK, 2 * E), lambda j: (j + CBLK, 0)),
            pl.BlockSpec((BK, 2 * E), lambda j: (j + 2 * CBLK, 0)),
            pl.BlockSpec((BK, 2 * E), lambda j: (j + 3 * CBLK, 0)),
            pl.BlockSpec((K_TIME, E * E), const),
            pl.BlockSpec((K_CAT, E * E), const),
            pl.BlockSpec((E, K_TIME), const),
            pl.BlockSpec((E, K_CAT), const),
            pl.BlockSpec((E, E * E), const),
            pl.BlockSpec((E, E * E), const),
        ],
        out_specs=out_specs,
        out_shape=out_shape,
    )(tcol, ccol, g, g, g, g, WtT, WcT, trtT, trcT, rep, tile)


def kernel(user_time_W, user_cat_W, POI_time_W, POI_cat_W, time_tr_W,
           time_proj_W, cat_tr_W, cat_proj_W, pos_u, pos_t, pos_p, pos_c,
           neg_u, neg_t, neg_p, neg_u2, neg_c, neg_p2, NS):
    i32 = jnp.int32
    iu_t = jnp.concatenate([pos_u, neg_u.reshape(-1)]).astype(i32)
    iu_c = jnp.concatenate([pos_u, neg_u2.reshape(-1)]).astype(i32)
    ip_t = jnp.concatenate([pos_p, neg_p.reshape(-1)]).astype(i32)
    ip_c = jnp.concatenate([pos_p, neg_p2.reshape(-1)]).astype(i32)
    idx_all = jnp.stack([iu_t, iu_c, ip_t, ip_c]).reshape(
        4, NCH, CB).transpose(1, 0, 2).reshape(-1)
    tcol = jnp.concatenate([pos_t, neg_t.reshape(-1)]).astype(i32).reshape(-1, 1)
    ccol = jnp.concatenate([pos_c, neg_c.reshape(-1)]).astype(i32).reshape(-1, 1)

    all4 = _pack_tables(user_time_W, user_cat_W, POI_time_W, POI_cat_W)
    gs = [_sc_gather(all4, idx_all, c * 4 * CB, 4 * CB) for c in range(NCH)]

    bf16 = jnp.bfloat16
    WtT = time_proj_W.astype(bf16)
    WcT = cat_proj_W.astype(bf16)
    trtT = time_tr_W.T.astype(bf16)
    trcT = cat_tr_W.T.astype(bf16)
    jj = jnp.arange(E * E)
    rr = jnp.arange(E)
    rep = (jj[None, :] // E == rr[:, None]).astype(bf16)
    tile = (jj[None, :] % E == rr[:, None]).astype(bf16)

    neg2d = None
    pos2d = None
    for c in range(NCH):
        outs = _tc_score_chunk(c, tcol, ccol, gs[c], WtT, WcT, trtT, trcT,
                               rep, tile)
        if c == 0:
            pos2d, nn = outs
            neg2d = nn
        else:
            (nn,) = outs
            neg2d = neg2d + nn
    pos = pos2d.reshape(-1)
    neg = neg2d[:neg_u.shape[0], 0]
    return (pos, neg)


# speedup vs baseline: 1.0125x; 1.0125x over previous
import functools

import jax
import jax.numpy as jnp
from jax import lax
from jax.experimental import pallas as pl
from jax.experimental.pallas import tpu as pltpu
from jax.experimental.pallas import tpu_sc as plsc

E = 64
BK = 1024
B = 4096
NTOT = 6 * B
K_TIME = 168
K_CAT = 400
CHUNK = 128
NBUF = 4
BP = 2000
NCH = 4
CB = NTOT // NCH
CBLK = CB // BK


def _pack_tables(user_time_W, user_cat_W, POI_time_W, POI_cat_W):
    a = jnp.concatenate([user_time_W, user_cat_W], axis=1)
    b = jnp.concatenate([POI_time_W, POI_cat_W], axis=1)
    abits = lax.bitcast_convert_type(a, jnp.int32) + 0x8000
    bbits = lax.bitcast_convert_type(b, jnp.int32) + 0x8000
    hi = jnp.bitwise_and(abits, -65536)
    lo = lax.shift_right_logical(bbits, 16)
    return jnp.bitwise_or(hi, lo)


def _sc_gather(all4, idx_all, base_row, n_rows):
    info = plsc.get_sparse_core_info()
    nw = info.num_cores * info.num_subcores
    bpw = n_rows // nw
    nchunk = bpw // CHUNK
    mesh = plsc.VectorSubcoreMesh(core_axis_name="c", subcore_axis_name="s")
    scratch = [pltpu.VMEM((CHUNK,), jnp.int32) for _ in range(NBUF)]
    scratch += [pltpu.VMEM((NBUF, CHUNK, 2 * E), jnp.int32),
                pltpu.SemaphoreType.DMA]

    @functools.partial(
        pl.kernel, mesh=mesh,
        out_type=jax.ShapeDtypeStruct((n_rows, 2 * E), jnp.int32),
        scratch_types=scratch)
    def gk(tab, idx, out, *sc):
        idxv = sc[:NBUF]
        rows = sc[NBUF]
        sem = sc[NBUF + 1]
        wid = lax.axis_index("s") * info.num_cores + lax.axis_index("c")
        base = wid * bpw
        descs = [None] * nchunk

        def writeout(k):
            pltpu.sync_copy(rows.at[k % NBUF],
                            out.at[pl.ds(base + k * CHUNK, CHUNK)])

        for k in range(nchunk):
            if k >= NBUF:
                descs[k - NBUF].wait()
                writeout(k - NBUF)
            pltpu.sync_copy(
                idx.at[pl.ds(base_row + base + k * CHUNK, CHUNK)],
                idxv[k % NBUF])
            descs[k] = pltpu.async_copy(tab.at[idxv[k % NBUF]],
                                        rows.at[k % NBUF], sem)
        for k in range(nchunk - NBUF, nchunk):
            descs[k].wait()
            writeout(k)

    return gk(all4, idx_all)


def _log_sigmoid(x):
    return -(jnp.maximum(-x, 0.0) + jnp.log(1.0 + jnp.exp(-jnp.abs(x))))


def _tc_score_chunk(c, tcol, ccol, g, WtT, WcT, trtT, trcT, rep, tile):
    bps = B // BK
    has_pos = c == 0

    def body(t_ref, c_ref, gua_ref, gub_ref, gpa_ref, gpb_ref,
             wtT_ref, wcT_ref, trtT_ref, trcT_ref, rep_ref, tile_ref,
             *out_refs):
        jl = pl.program_id(0)
        jg = c * CBLK + jl
        if has_pos:
            pos_ref, neg_ref = out_refs
        else:
            (neg_ref,) = out_refs

        def score(u, p, wT, trT, idx, K):
            prep = jnp.dot(p, rep_ref[...], preferred_element_type=jnp.float32)
            u2 = jnp.concatenate([u, u], axis=1)
            v = jnp.concatenate(
                [(prep[:, k * 128:(k + 1) * 128] * u2).astype(jnp.bfloat16)
                 for k in range(E * E // 128)], axis=1)
            cdims = (((1,), (1,)), ((), ()))
            y = lax.dot_general(v, wT, cdims,
                                preferred_element_type=jnp.float32)
            y = y + jnp.dot(p, trT, preferred_element_type=jnp.float32)
            oh = (lax.broadcasted_iota(jnp.int32, (BK, K), 1) == idx)
            return jnp.sum(jnp.where(oh, y, 0.0), axis=1, keepdims=True)

        def unpack_user(x):
            return lax.bitcast_convert_type(
                jnp.bitwise_and(x, -65536), jnp.float32).astype(jnp.bfloat16)

        def unpack_poi(x):
            return lax.bitcast_convert_type(
                lax.shift_left(x, 16), jnp.float32).astype(jnp.bfloat16)

        ut = unpack_user(gua_ref[:, :E])
        uc = unpack_user(gub_ref[:, E:])
        pt = unpack_poi(gpa_ref[:, :E])
        pc = unpack_poi(gpb_ref[:, E:])
        s_t = score(ut, pt, wtT_ref[...], trtT_ref[...], t_ref[...], K_TIME)
        s_c = score(uc, pc, wcT_ref[...], trcT_ref[...], c_ref[...], K_CAT)
        set_id = jg // bps

        @pl.when(jl == 0)
        def _():
            neg_ref[...] = jnp.zeros_like(neg_ref)

        if has_pos:
            @pl.when(set_id == 0)
            def _():
                pos_ref[...] = -(_log_sigmoid(s_t) + _log_sigmoid(s_c))

        @pl.when(set_id > 0)
        def _():
            part = jnp.sum(_log_sigmoid(-s_t) + _log_sigmoid(-s_c))
            rr = lax.broadcasted_iota(jnp.int32, (8, 128), 0)
            cc = lax.broadcasted_iota(jnp.int32, (8, 128), 1)
            m = (rr == (set_id - 1)) & (cc == 0)
            neg_ref[...] = neg_ref[...] + jnp.where(m, -part, 0.0)

    const = lambda j: (0, 0)
    srow = lambda j: (c * CBLK + j, 0)
    out_specs = [pl.BlockSpec((8, 128), const)]
    out_shape = [jax.ShapeDtypeStruct((8, 128), jnp.float32)]
    if has_pos:
        out_specs = [pl.BlockSpec((BK, 1), lambda j: (jnp.minimum(j, bps - 1), 0))] + out_specs
        out_shape = [jax.ShapeDtypeStruct((B, 1), jnp.float32)] + out_shape
    return pl.pallas_call(
        body,
        grid=(CBLK,),
        in_specs=[
            pl.BlockSpec((BK, 1), srow),
            pl.BlockSpec((BK, 1), srow),
            pl.BlockSpec((BK, 2 * E), lambda j: (j, 0)),
            pl.BlockSpec((BK, 2 * E), lambda j: (j + CBLK, 0)),
            pl.BlockSpec((BK, 2 * E), lambda j: (j + 2 * CBLK, 0)),
            pl.BlockSpec((BK, 2 * E), lambda j: (j + 3 * CBLK, 0)),
            pl.BlockSpec((K_TIME, E * E), const),
            pl.BlockSpec((K_CAT, E * E), const),
            pl.BlockSpec((E, K_TIME), const),
            pl.BlockSpec((E, K_CAT), const),
            pl.BlockSpec((E, E * E), const),
            pl.BlockSpec((E, E * E), const),
        ],
        out_specs=out_specs,
        out_shape=out_shape,
    )(tcol, ccol, g, g, g, g, WtT, WcT, trtT, trcT, rep, tile)


def kernel(user_time_W, user_cat_W, POI_time_W, POI_cat_W, time_tr_W,
           time_proj_W, cat_tr_W, cat_proj_W, pos_u, pos_t, pos_p, pos_c,
           neg_u, neg_t, neg_p, neg_u2, neg_c, neg_p2, NS):
    i32 = jnp.int32
    iu_t = jnp.concatenate([pos_u, neg_u.reshape(-1)]).astype(i32)
    iu_c = jnp.concatenate([pos_u, neg_u2.reshape(-1)]).astype(i32)
    ip_t = jnp.concatenate([pos_p, neg_p.reshape(-1)]).astype(i32)
    ip_c = jnp.concatenate([pos_p, neg_p2.reshape(-1)]).astype(i32)
    idx_all = jnp.stack([iu_t, iu_c, ip_t, ip_c]).reshape(
        4, NCH, CB).transpose(1, 0, 2).reshape(-1)
    tcol = jnp.concatenate([pos_t, neg_t.reshape(-1)]).astype(i32).reshape(-1, 1)
    ccol = jnp.concatenate([pos_c, neg_c.reshape(-1)]).astype(i32).reshape(-1, 1)

    all4 = _pack_tables(user_time_W, user_cat_W, POI_time_W, POI_cat_W)
    gs = [_sc_gather(all4, idx_all, c * 4 * CB, 4 * CB) for c in range(NCH)]

    bf16 = jnp.bfloat16
    WtT = time_proj_W.astype(bf16)
    WcT = cat_proj_W.astype(bf16)
    trtT = time_tr_W.T.astype(bf16)
    trcT = cat_tr_W.T.astype(bf16)
    jj = jnp.arange(E * E)
    rr = jnp.arange(E)
    rep = (jj[None, :] // E == rr[:, None]).astype(bf16)
    tile = (jj[None, :] % E == rr[:, None]).astype(bf16)

    neg2d = None
    pos2d = None
    for c in range(NCH):
        outs = _tc_score_chunk(c, tcol, ccol, gs[c], WtT, WcT, trtT, trcT,
                               rep, tile)
        if c == 0:
            pos2d, nn = outs
            neg2d = nn
        else:
            (nn,) = outs
            neg2d = neg2d + nn
    pos = pos2d.reshape(-1)
    neg = neg2d[:neg_u.shape[0], 0]
    return (pos, neg)
